# Initial kernel scaffold; baseline (speedup 1.0000x reference)
import jax
import jax.numpy as jnp
from jax.experimental import pallas as pl


def kernel(x, edge_index, W1a, b1a, W2a, b2a, W1b, b1b, W2b, b2b, W1c, b1c, W2c, b2c):
    raise NotImplementedError("write your pallas kernel here")


# speedup vs baseline: 4.6327x; 4.6327x over previous
import functools

import jax
import jax.numpy as jnp
from jax import lax
import jax.experimental.pallas as pl
from jax.experimental.pallas import tpu as pltpu
from jax.experimental.pallas import tpu_sc as plsc

N = 100000
E = 6400000
D = 64

NPB = 512
NB = 196
NPAD = NB * NPB
NW = 32
CAP = 2048
EW = E // NW
W_E = 2048
NWIN = (EW + W_E - 1) // W_E
CH = 128
TOTP = (NB + 1) * NW * CAP
NEG = -1.0e30

_mesh = plsc.VectorSubcoreMesh(
    core_axis_name="c", subcore_axis_name="s", num_cores=2, num_subcores=16)


def _wid():
  return lax.axis_index("s") * 2 + lax.axis_index("c")


def _vec_at(ref, i):
  v = plsc.load_gather(ref, [lax.full((16,), i, jnp.int32)])
  return jnp.max(v)


@functools.partial(
    pl.kernel,
    out_type=(jax.ShapeDtypeStruct((TOTP,), jnp.int32),
              jax.ShapeDtypeStruct((NW, 224), jnp.int32)),
    mesh=_mesh,
    compiler_params=pltpu.CompilerParams(needs_layout_passes=False, use_tc_tiling_on_sc=False),
    scratch_types=[
        pltpu.VMEM((W_E,), jnp.int32),
        pltpu.VMEM((W_E,), jnp.int32),
        pltpu.VMEM((224,), jnp.int32),
        pltpu.VMEM((16, 128), jnp.int32),
        pltpu.VMEM((16, 128), jnp.int32),
        pltpu.SemaphoreType.DMA,
    ],
)
def _partition_k(src_hbm, dst_hbm, pairs_hbm, counts_hbm,
                 sv, dv, cnt_v, idxb, pairb, sem):
  w = _wid()
  iota = lax.iota(jnp.int32, 16)
  r0, _ = plsc.scan_count(iota)

  def zero_body(k, _):
    cnt_v[pl.ds(k * 16, 16)] = jnp.zeros((16,), jnp.int32)
    return _
  lax.fori_loop(0, 14, zero_body, None)

  def win_body(win, _):
    base = w * EW + win * W_E
    pltpu.sync_copy(src_hbm.at[pl.ds(base, W_E)], sv)
    pltpu.sync_copy(dst_hbm.at[pl.ds(base, W_E)], dv)

    def vec_body(i, _):
      pos = i * 16 + iota
      valid = (win * W_E + pos) < EW
      s = sv[pl.ds(i * 16, 16)]
      d = dv[pl.ds(i * 16, 16)]
      b = jnp.where(valid, lax.shift_right_logical(d, 9), NB)
      rank, last = plsc.scan_count(b)
      old = plsc.load_gather(cnt_v, [b])
      rk = rank - r0
      slot = jnp.minimum(old + rk, CAP - 1)
      plsc.store_scatter(cnt_v, [b], slot + 1, mask=last)
      gslot = b * (NW * CAP) + w * CAP + slot
      packed = jnp.bitwise_or(
          s, lax.shift_left(jnp.bitwise_and(d, NPB - 1), 17))
      hi = lax.shift_right_logical(pos, 7)
      lo = jnp.bitwise_and(pos, 127)
      plsc.store_scatter(idxb, [hi, lo], gslot)
      plsc.store_scatter(pairb, [hi, lo], packed)
      return _
    lax.fori_loop(0, W_E // 16, vec_body, None)

    cps = [pltpu.async_copy(pairb.at[j], pairs_hbm.at[idxb.at[j]], sem)
           for j in range(16)]
    for cp in cps:
      cp.wait()
    return _
  lax.fori_loop(0, NWIN, win_body, None)

  pltpu.sync_copy(cnt_v, counts_hbm.at[w])


@functools.partial(
    pl.kernel,
    out_type=jax.ShapeDtypeStruct((NPAD * D,), jnp.float32),
    mesh=_mesh,
    compiler_params=pltpu.CompilerParams(needs_layout_passes=False, use_tc_tiling_on_sc=False),
    scratch_types=[
        pltpu.VMEM(((NPB + 1) * D,), jnp.float32),
        pltpu.VMEM((CH, D), jnp.float32),
        pltpu.VMEM((CH,), jnp.int32),
        pltpu.VMEM((CH,), jnp.int32),
        pltpu.VMEM((CH,), jnp.int32),
        pltpu.VMEM((NW * 224,), jnp.int32),
        pltpu.SemaphoreType.DMA,
    ],
)
def _segmax_k(y_hbm, pairs_hbm, counts_hbm,
              m_hbm, acc, rows2d, pc, gidx, dloc, cnts, sem):
  w = _wid()
  iota = lax.iota(jnp.int32, 16)
  pltpu.sync_copy(counts_hbm, cnts)

  for r in range(7):
    b = w + 32 * r

    @pl.when(b < NB)
    def _():
      def init_body(k, _):
        acc[pl.ds(k * 16, 16)] = lax.full((16,), NEG, jnp.float32)
        return _
      lax.fori_loop(0, (NPB + 1) * D // 16, init_body, None)

      def t_body(t, _):
        ct = _vec_at(cnts, t * 224 + b)
        nch = lax.shift_right_logical(ct + (CH - 1), 7)

        def c_body(c, _):
          rbase = (b * NW + t) * CAP + c * CH
          pltpu.sync_copy(pairs_hbm.at[pl.ds(rbase, CH)], pc)

          def s_body(i, _):
            pos = c * CH + i * 16 + iota
            valid = pos < ct
            p = pc[pl.ds(i * 16, 16)]
            s = jnp.bitwise_and(p, (1 << 17) - 1)
            dl = jnp.bitwise_and(lax.shift_right_logical(p, 17), NPB - 1)
            gidx[pl.ds(i * 16, 16)] = jnp.where(valid, s, iota)
            dloc[pl.ds(i * 16, 16)] = jnp.where(valid, dl, NPB)
            return _
          lax.fori_loop(0, CH // 16, s_body, None)

          pltpu.async_copy(y_hbm.at[gidx], rows2d, sem).wait()

          def e_body(e, _):
            dl = _vec_at(dloc, e)
            off = dl * D
            for g in range(4):
              a = acc[pl.ds(off + g * 16, 16)]
              v = rows2d[e, pl.ds(g * 16, 16)]
              acc[pl.ds(off + g * 16, 16)] = jnp.maximum(a, v)
            return _
          lax.fori_loop(0, CH, e_body, None)
          return _
        lax.fori_loop(0, nch, c_body, None)
        return _
      lax.fori_loop(0, NW, t_body, None)

      pltpu.sync_copy(acc.at[pl.ds(0, NPB * D)],
                      m_hbm.at[pl.ds(b * NPB * D, NPB * D)])


_TBLK = 1024
_GRID = NPAD // _TBLK


def _tc_y0(xpad, w1p):
  def body(x_ref, w_ref, o_ref):
    o_ref[...] = jnp.dot(x_ref[...], w_ref[...],
                         preferred_element_type=jnp.float32)
  return pl.pallas_call(
      body,
      grid=(_GRID,),
      in_specs=[pl.BlockSpec((_TBLK, 128), lambda i: (i, 0)),
                pl.BlockSpec((128, D), lambda i: (0, 0))],
      out_specs=pl.BlockSpec((_TBLK, D), lambda i: (i, 0)),
      out_shape=jax.ShapeDtypeStruct((NPAD, D), jnp.float32),
  )(xpad, w1p)


def _tc_mid(m, y, b1, w2p, b2p, w1n):
  def body(m_ref, y_ref, b1_ref, w2_ref, b2_ref, w1_ref, o_ref):
    mm = m_ref[...]
    agg = jnp.where(mm > -1.0e29, mm - y_ref[...] + b1_ref[0:1, :], 0.0)
    h = jnp.maximum(
        jnp.dot(agg, w2_ref[...], preferred_element_type=jnp.float32)
        + b2_ref[0:1, :], 0.0)
    o_ref[...] = jnp.dot(h, w1_ref[...], preferred_element_type=jnp.float32)
  return pl.pallas_call(
      body,
      grid=(_GRID,),
      in_specs=[pl.BlockSpec((_TBLK, D), lambda i: (i, 0)),
                pl.BlockSpec((_TBLK, D), lambda i: (i, 0)),
                pl.BlockSpec((8, D), lambda i: (0, 0)),
                pl.BlockSpec((D, 128), lambda i: (0, 0)),
                pl.BlockSpec((8, 128), lambda i: (0, 0)),
                pl.BlockSpec((128, D), lambda i: (0, 0))],
      out_specs=pl.BlockSpec((_TBLK, D), lambda i: (i, 0)),
      out_shape=jax.ShapeDtypeStruct((NPAD, D), jnp.float32),
  )(m, y, b1, w2p, b2p, w1n)


def _tc_fin(m, y, b1, w2p, b2p):
  def body(m_ref, y_ref, b1_ref, w2_ref, b2_ref, o_ref):
    mm = m_ref[...]
    agg = jnp.where(mm > -1.0e29, mm - y_ref[...] + b1_ref[0:1, :], 0.0)
    h = jnp.dot(agg, w2_ref[...], preferred_element_type=jnp.float32) \
        + b2_ref[0:1, :]
    o_ref[...] = jax.nn.sigmoid(h)
  return pl.pallas_call(
      body,
      grid=(_GRID,),
      in_specs=[pl.BlockSpec((_TBLK, D), lambda i: (i, 0)),
                pl.BlockSpec((_TBLK, D), lambda i: (i, 0)),
                pl.BlockSpec((8, D), lambda i: (0, 0)),
                pl.BlockSpec((D, 128), lambda i: (0, 0)),
                pl.BlockSpec((8, 128), lambda i: (0, 0))],
      out_specs=pl.BlockSpec((_TBLK, 128), lambda i: (i, 0)),
      out_shape=jax.ShapeDtypeStruct((NPAD, 128), jnp.float32),
  )(m, y, b1, w2p, b2p)


def kernel(x, edge_index, W1a, b1a, W2a, b2a, W1b, b1b, W2b, b2b,
           W1c, b1c, W2c, b2c):
  src = edge_index[0].astype(jnp.int32)
  dst = edge_index[1].astype(jnp.int32)
  pad_e = NWIN * W_E - EW
  src = jnp.pad(src, (0, pad_e))
  dst = jnp.pad(dst, (0, pad_e))

  pairs_f, counts = _partition_k(src, dst)
  counts_f = counts.reshape(-1)

  xpad = jnp.pad(x, ((0, NPAD - N), (0, 128 - 3)))

  def pw1(w):
    return jnp.pad(w, ((0, 128 - 3), (0, 0)))

  def pw2(w):
    return jnp.pad(w, ((0, 0), (0, 128 - w.shape[1])))

  def pb2(bv):
    return jnp.broadcast_to(jnp.pad(bv, (0, 128 - bv.shape[0])), (8, 128))

  y1 = _tc_y0(xpad, pw1(W1a))
  m1 = _segmax_k(y1, pairs_f, counts_f).reshape(NPAD, D)
  y2 = _tc_mid(m1, y1, jnp.broadcast_to(b1a, (8, D)), pw2(W2a), pb2(b2a), pw1(W1b))
  m2 = _segmax_k(y2, pairs_f, counts_f).reshape(NPAD, D)
  y3 = _tc_mid(m2, y2, jnp.broadcast_to(b1b, (8, D)), pw2(W2b), pb2(b2b), pw1(W1c))
  m3 = _segmax_k(y3, pairs_f, counts_f).reshape(NPAD, D)
  outp = _tc_fin(m3, y3, jnp.broadcast_to(b1c, (8, D)), pw2(W2c), pb2(b2c))
  return outp[:N, :1]


# speedup vs baseline: 6.0986x; 1.3164x over previous
import functools

import jax
import jax.numpy as jnp
from jax import lax
import jax.experimental.pallas as pl
from jax.experimental.pallas import tpu as pltpu
from jax.experimental.pallas import tpu_sc as plsc

N = 100000
E = 6400000
D = 64

NPB = 512
NB = 196
NPAD = NB * NPB
NW = 32
EW = E // NW
W_E = 2048
NWIN = (EW + W_E - 1) // W_E
WPS = 12
NSW = (NWIN + WPS - 1) // WPS
BLK = 256
CAP = NSW * BLK
CW = 2048
TOTP = NB * NW * CAP
NTB = NW * NSW
NEG = -1.0e30

_mesh = plsc.VectorSubcoreMesh(
    core_axis_name="c", subcore_axis_name="s", num_cores=2, num_subcores=16)


def _wid():
  return lax.axis_index("s") * 2 + lax.axis_index("c")


def _vec_at(ref, i):
  v = plsc.load_gather(ref, [lax.full((16,), i, jnp.int32)])
  return jnp.max(v)


@functools.partial(
    pl.kernel,
    out_type=(jax.ShapeDtypeStruct((TOTP,), jnp.int32),
              jax.ShapeDtypeStruct((NW, CW), jnp.int32)),
    mesh=_mesh,
    compiler_params=pltpu.CompilerParams(needs_layout_passes=False,
                                         use_tc_tiling_on_sc=False),
    scratch_types=[
        pltpu.VMEM((W_E,), jnp.int32),
        pltpu.VMEM((W_E,), jnp.int32),
        pltpu.VMEM((224,), jnp.int32),
        pltpu.VMEM(((NB + 1) * BLK,), jnp.int32),
        pltpu.VMEM((CW,), jnp.int32),
        pltpu.SemaphoreType.DMA,
    ],
)
def _partition_k(src_hbm, dst_hbm, pairs_hbm, counts_hbm,
                 sv, dv, cnt_v, stg, crow, sem):
  w = _wid()
  iota = lax.iota(jnp.int32, 16)
  r0, _ = plsc.scan_count(iota)

  def sw_body(sw, _):
    def zero_body(k, _):
      cnt_v[pl.ds(k * 16, 16)] = jnp.zeros((16,), jnp.int32)
      return _
    lax.fori_loop(0, 14, zero_body, None)

    nwin = jnp.minimum(WPS, NWIN - sw * WPS)

    def win_body(win, _):
      g = sw * WPS + win
      base = w * EW + g * W_E
      pltpu.sync_copy(src_hbm.at[pl.ds(base, W_E)], sv)
      pltpu.sync_copy(dst_hbm.at[pl.ds(base, W_E)], dv)

      def vec_body(i, _):
        pos = i * 16 + iota
        valid = (g * W_E + pos) < EW
        s = sv[pl.ds(i * 16, 16)]
        d = dv[pl.ds(i * 16, 16)]
        b = jnp.where(valid, lax.shift_right_logical(d, 9), NB)
        rank, last = plsc.scan_count(b)
        old = plsc.load_gather(cnt_v, [b])
        slot = jnp.minimum(old + (rank - r0), BLK - 1)
        plsc.store_scatter(cnt_v, [b], jnp.minimum(slot + 1, BLK), mask=last)
        packed = jnp.bitwise_or(
            s, lax.shift_left(jnp.bitwise_and(d, NPB - 1), 17))
        plsc.store_scatter(stg, [b * BLK + slot], packed)
        return _
      lax.fori_loop(0, W_E // 16, vec_body, None)
      return _
    lax.fori_loop(0, nwin, win_body, None)

    cps = [pltpu.async_copy(
        stg.at[pl.ds(jb * BLK, BLK)],
        pairs_hbm.at[pl.ds((jb * NW + w) * CAP + sw * BLK, BLK)], sem)
        for jb in range(NB)]
    for cp in cps:
      cp.wait()

    def crow_body(k, _):
      crow[pl.ds(sw * 224 + k * 16, 16)] = cnt_v[pl.ds(k * 16, 16)]
      return _
    lax.fori_loop(0, 14, crow_body, None)
    return _
  lax.fori_loop(0, NSW, sw_body, None)

  pltpu.sync_copy(crow, counts_hbm.at[w])


@functools.partial(
    pl.kernel,
    out_type=jax.ShapeDtypeStruct((NPAD * D,), jnp.float32),
    mesh=_mesh,
    compiler_params=pltpu.CompilerParams(needs_layout_passes=False,
                                         use_tc_tiling_on_sc=False),
    scratch_types=[
        pltpu.VMEM(((NPB + 1) * D,), jnp.float32),
        pltpu.VMEM((4, 128, D), jnp.float32),
        pltpu.VMEM((2, BLK), jnp.int32),
        pltpu.VMEM((2, 2, 128), jnp.int32),
        pltpu.VMEM((2, BLK), jnp.int32),
        pltpu.VMEM((NTB,), jnp.int32),
        pltpu.SemaphoreType.DMA,
        pltpu.SemaphoreType.DMA,
    ],
)
def _segmax_k(y_hbm, pairs_hbm, counts_hbm,
              m_hbm, acc, rows, pc, gidx, dloc, cnts, sem, sem2):
  w = _wid()
  iota = lax.iota(jnp.int32, 16)

  for r in range(7):
    b = w + 32 * r

    @pl.when(b < NB)
    def _():
      def init_body(k, _):
        acc[pl.ds(k * 16, 16)] = lax.full((16,), NEG, jnp.float32)
        return _
      lax.fori_loop(0, (NPB + 1) * D // 16, init_body, None)

      ccps = [pltpu.async_copy(
          counts_hbm.at[pl.ds((s * 224 + b) * NW, NW)],
          cnts.at[pl.ds(s * NW, NW)], sem2) for s in range(NSW)]
      for cp in ccps:
        cp.wait()

      pltpu.sync_copy(pairs_hbm.at[pl.ds(b * NW * CAP, BLK)], pc.at[0])

      def blk_body(j, carry):
        t, sw, pne = carry
        par = j & 1
        sw1 = sw + 1
        wrap = sw1 == NSW
        nt = jnp.where(wrap, t + 1, t)
        nsw = jnp.where(wrap, 0, sw1)

        ct = _vec_at(cnts, jnp.minimum(sw * NW + t, NTB - 1))
        n16 = jnp.minimum(jnp.bitwise_and(ct + 15, ~15), BLK)

        @pl.when(j < NTB)
        def _fire():
          for i in range(BLK // 16):
            p = pc[par, pl.ds(i * 16, 16)]
            pos = i * 16 + iota
            valid = pos < ct
            s = jnp.bitwise_and(p, (1 << 17) - 1)
            dl = jnp.bitwise_and(lax.shift_right_logical(p, 17), NPB - 1)
            gidx[par, i // 8, pl.ds((i % 8) * 16, 16)] = \
                jnp.where(valid, s, iota)
            dloc[par, pl.ds(i * 16, 16)] = jnp.where(valid, dl, NPB)
          pltpu.async_copy(y_hbm.at[gidx.at[par, 0]], rows.at[par * 2], sem)

          @pl.when(ct > 128)
          def _():
            pltpu.async_copy(y_hbm.at[gidx.at[par, 1]],
                             rows.at[par * 2 + 1], sem)

          @pl.when(j + 1 < NTB)
          def _():
            nbase = (b * NW + nt) * CAP + nsw * BLK
            pltpu.async_copy(pairs_hbm.at[pl.ds(nbase, BLK)],
                             pc.at[1 - par], sem2)

        @pl.when(j > 0)
        def _accum():
          pp = 1 - par

          def grp_body(gi, _):
            dv16 = dloc[pp, pl.ds(gi * 16, 16)]
            cc = pp * 2 + lax.shift_right_logical(gi, 3)
            ebase = jnp.bitwise_and(gi, 7) * 16
            for lane in range(16):
              dl = dv16[lane]
              off = dl * D
              for gch in range(4):
                a = acc[pl.ds(off + gch * 16, 16)]
                v = rows[cc, ebase + lane, pl.ds(gch * 16, 16)]
                acc[pl.ds(off + gch * 16, 16)] = jnp.maximum(a, v)
            return _
          lax.fori_loop(0, lax.shift_right_logical(pne, 4), grp_body, None)

        @pl.when(j < NTB)
        def _wait():
          pltpu.make_async_copy(y_hbm.at[gidx.at[par, 0]],
                                rows.at[par * 2], sem).wait()

          @pl.when(ct > 128)
          def _():
            pltpu.make_async_copy(y_hbm.at[gidx.at[par, 1]],
                                  rows.at[par * 2 + 1], sem).wait()

          @pl.when(j + 1 < NTB)
          def _():
            nbase = (b * NW + nt) * CAP + nsw * BLK
            pltpu.make_async_copy(pairs_hbm.at[pl.ds(nbase, BLK)],
                                  pc.at[1 - par], sem2).wait()

        return (nt, nsw, n16)

      lax.fori_loop(0, NTB + 1, blk_body,
                    (jnp.int32(0), jnp.int32(0), jnp.int32(0)))

      pltpu.sync_copy(acc.at[pl.ds(0, NPB * D)],
                      m_hbm.at[pl.ds(b * NPB * D, NPB * D)])


_TBLK = 1024
_GRID = NPAD // _TBLK


def _tc_y0(xpad, w1p):
  def body(x_ref, w_ref, o_ref):
    o_ref[...] = jnp.dot(x_ref[...], w_ref[...],
                         preferred_element_type=jnp.float32)
  return pl.pallas_call(
      body,
      grid=(_GRID,),
      in_specs=[pl.BlockSpec((_TBLK, 128), lambda i: (i, 0)),
                pl.BlockSpec((128, D), lambda i: (0, 0))],
      out_specs=pl.BlockSpec((_TBLK, D), lambda i: (i, 0)),
      out_shape=jax.ShapeDtypeStruct((NPAD, D), jnp.float32),
  )(xpad, w1p)


def _tc_mid(m, y, b1, w2p, b2p, w1n):
  def body(m_ref, y_ref, b1_ref, w2_ref, b2_ref, w1_ref, o_ref):
    mm = m_ref[...]
    agg = jnp.where(mm > -1.0e29, mm - y_ref[...] + b1_ref[0:1, :], 0.0)
    h = jnp.maximum(
        jnp.dot(agg, w2_ref[...], preferred_element_type=jnp.float32)
        + b2_ref[0:1, :], 0.0)
    o_ref[...] = jnp.dot(h, w1_ref[...], preferred_element_type=jnp.float32)
  return pl.pallas_call(
      body,
      grid=(_GRID,),
      in_specs=[pl.BlockSpec((_TBLK, D), lambda i: (i, 0)),
                pl.BlockSpec((_TBLK, D), lambda i: (i, 0)),
                pl.BlockSpec((8, D), lambda i: (0, 0)),
                pl.BlockSpec((D, 128), lambda i: (0, 0)),
                pl.BlockSpec((8, 128), lambda i: (0, 0)),
                pl.BlockSpec((128, D), lambda i: (0, 0))],
      out_specs=pl.BlockSpec((_TBLK, D), lambda i: (i, 0)),
      out_shape=jax.ShapeDtypeStruct((NPAD, D), jnp.float32),
  )(m, y, b1, w2p, b2p, w1n)


def _tc_fin(m, y, b1, w2p, b2p):
  def body(m_ref, y_ref, b1_ref, w2_ref, b2_ref, o_ref):
    mm = m_ref[...]
    agg = jnp.where(mm > -1.0e29, mm - y_ref[...] + b1_ref[0:1, :], 0.0)
    h = jnp.dot(agg, w2_ref[...], preferred_element_type=jnp.float32) \
        + b2_ref[0:1, :]
    o_ref[...] = jax.nn.sigmoid(h)
  return pl.pallas_call(
      body,
      grid=(_GRID,),
      in_specs=[pl.BlockSpec((_TBLK, D), lambda i: (i, 0)),
                pl.BlockSpec((_TBLK, D), lambda i: (i, 0)),
                pl.BlockSpec((8, D), lambda i: (0, 0)),
                pl.BlockSpec((D, 128), lambda i: (0, 0)),
                pl.BlockSpec((8, 128), lambda i: (0, 0))],
      out_specs=pl.BlockSpec((_TBLK, 128), lambda i: (i, 0)),
      out_shape=jax.ShapeDtypeStruct((NPAD, 128), jnp.float32),
  )(m, y, b1, w2p, b2p)


def kernel(x, edge_index, W1a, b1a, W2a, b2a, W1b, b1b, W2b, b2b,
           W1c, b1c, W2c, b2c):
  src = edge_index[0].astype(jnp.int32)
  dst = edge_index[1].astype(jnp.int32)
  pad_e = NWIN * W_E - EW
  src = jnp.pad(src, (0, pad_e))
  dst = jnp.pad(dst, (0, pad_e))

  pairs_f, counts = _partition_k(src, dst)
  counts_f = counts.T.reshape(-1)

  xpad = jnp.pad(x, ((0, NPAD - N), (0, 128 - 3)))

  def pw1(w):
    return jnp.pad(w, ((0, 128 - 3), (0, 0)))

  def pw2(w):
    return jnp.pad(w, ((0, 0), (0, 128 - w.shape[1])))

  def pb2(bv):
    return jnp.broadcast_to(jnp.pad(bv, (0, 128 - bv.shape[0])), (8, 128))

  y1 = _tc_y0(xpad, pw1(W1a))
  m1 = _segmax_k(y1, pairs_f, counts_f).reshape(NPAD, D)
  y2 = _tc_mid(m1, y1, jnp.broadcast_to(b1a, (8, D)), pw2(W2a), pb2(b2a), pw1(W1b))
  m2 = _segmax_k(y2, pairs_f, counts_f).reshape(NPAD, D)
  y3 = _tc_mid(m2, y2, jnp.broadcast_to(b1b, (8, D)), pw2(W2b), pb2(b2b), pw1(W1c))
  m3 = _segmax_k(y3, pairs_f, counts_f).reshape(NPAD, D)
  outp = _tc_fin(m3, y3, jnp.broadcast_to(b1c, (8, D)), pw2(W2c), pb2(b2c))
  return outp[:N, :1]
